# Initial kernel scaffold; baseline (speedup 1.0000x reference)
import jax
import jax.numpy as jnp
from jax.experimental import pallas as pl


def kernel(dayofweek, time, sex, age, month, day, W_dayofweek, W_time, W_sex, W_age, W_month, W_day):
    raise NotImplementedError("write your pallas kernel here")


# speedup vs baseline: 6.7569x; 6.7569x over previous
import functools

import jax
import jax.numpy as jnp
from jax import lax
from jax.experimental import pallas as pl
from jax.experimental.pallas import tpu as pltpu
from jax.experimental.pallas import tpu_sc as plsc

B = 16384
D = 64
NF = 6
NP = 3
PW = 2 * D
NC, NS = 2, 16
NW = NC * NS
R = B // NW
C = 128
NCH = R // C
L = 16

PAIR_VB = (24, 100, 31)


def kernel(dayofweek, time, sex, age, month, day,
           W_dayofweek, W_time, W_sex, W_age, W_month, W_day):
    def pair_table(Wa, Wb):
        va, vb = Wa.shape[0], Wb.shape[0]
        return jnp.concatenate(
            [jnp.repeat(Wa, vb, axis=0), jnp.tile(Wb, (va, 1))], axis=1)

    T0 = pair_table(W_dayofweek, W_time)
    T1 = pair_table(W_sex, W_age)
    T2 = pair_table(W_month, W_day)

    mesh = plsc.VectorSubcoreMesh(
        core_axis_name="c", subcore_axis_name="s",
        num_cores=NC, num_subcores=NS)

    @functools.partial(
        pl.kernel,
        out_type=jax.ShapeDtypeStruct((B, NF * D), jnp.float32),
        mesh=mesh,
        scratch_types=[
            pltpu.VMEM((NF * R,), jnp.int32),
            pltpu.VMEM((NP * R,), jnp.int32),
            pltpu.VMEM((C, NF * D), jnp.float32),
            pltpu.SemaphoreType.DMA,
        ],
    )
    def sck(i0, i1, i2, i3, i4, i5, t0, t1, t2,
            out, raw_v, cidx_v, asm_v, sem):
        wid = lax.axis_index("s") * NC + lax.axis_index("c")
        base = wid * R
        idxs = (i0, i1, i2, i3, i4, i5)
        tables = (t0, t1, t2)
        for f in range(NF):
            pltpu.sync_copy(idxs[f].at[pl.ds(base, R)],
                            raw_v.at[pl.ds(f * R, R)])
        for p in range(NP):
            vb = PAIR_VB[p]
            for j in range(R // L):
                ia = raw_v[pl.ds((2 * p) * R + j * L, L)]
                ib = raw_v[pl.ds((2 * p + 1) * R + j * L, L)]
                cidx_v[pl.ds(p * R + j * L, L)] = ia * vb + ib
        for c in range(NCH):
            copies = [
                pltpu.async_copy(
                    tables[p].at[cidx_v.at[pl.ds(p * R + c * C, C)]],
                    asm_v.at[:, pl.ds(p * PW, PW)], sem)
                for p in range(NP)]
            for cp in copies:
                cp.wait()
            pltpu.sync_copy(asm_v, out.at[pl.ds(base + c * C, C), :])

    return sck(dayofweek.astype(jnp.int32), time.astype(jnp.int32),
               sex.astype(jnp.int32), age.astype(jnp.int32),
               month.astype(jnp.int32), day.astype(jnp.int32),
               T0, T1, T2)


# speedup vs baseline: 7.0985x; 1.0506x over previous
import functools

import jax
import jax.numpy as jnp
from jax import lax
from jax.experimental import pallas as pl
from jax.experimental.pallas import tpu as pltpu
from jax.experimental.pallas import tpu_sc as plsc

B = 16384
D = 64
NF = 6
NP = 3
PW = 2 * D
NC, NS = 2, 16
NW = NC * NS
R = B // NW
C = 128
NCH = R // C
L = 16

PAIR_VB = (24, 100, 31)


def kernel(dayofweek, time, sex, age, month, day,
           W_dayofweek, W_time, W_sex, W_age, W_month, W_day):
    def pair_table(Wa, Wb):
        va, vb = Wa.shape[0], Wb.shape[0]
        return jnp.concatenate(
            [jnp.repeat(Wa, vb, axis=0), jnp.tile(Wb, (va, 1))], axis=1)

    T0 = pair_table(W_dayofweek, W_time)
    T1 = pair_table(W_sex, W_age)
    T2 = pair_table(W_month, W_day)

    mesh = plsc.VectorSubcoreMesh(
        core_axis_name="c", subcore_axis_name="s",
        num_cores=NC, num_subcores=NS)

    @functools.partial(
        pl.kernel,
        out_type=jax.ShapeDtypeStruct((B, NF * D), jnp.float32),
        mesh=mesh,
        scratch_types=[
            pltpu.VMEM((NF * R,), jnp.int32),
            pltpu.VMEM((NP * R,), jnp.int32),
            pltpu.VMEM((2, C, NF * D), jnp.float32),
            pltpu.SemaphoreType.DMA,
            pltpu.SemaphoreType.DMA,
            pltpu.SemaphoreType.DMA,
            pltpu.SemaphoreType.DMA,
        ],
    )
    def sck(i0, i1, i2, i3, i4, i5, t0, t1, t2,
            out, raw_v, cidx_v, asm_v, g0, g1, w0, w1):
        wid = lax.axis_index("s") * NC + lax.axis_index("c")
        base = wid * R
        idxs = (i0, i1, i2, i3, i4, i5)
        tables = (t0, t1, t2)
        gsems = (g0, g1)
        wsems = (w0, w1)
        stage = [pltpu.async_copy(idxs[f].at[pl.ds(base, R)],
                                  raw_v.at[pl.ds(f * R, R)], g0)
                 for f in range(NF)]
        for cp in stage:
            cp.wait()
        for p in range(NP):
            vb = PAIR_VB[p]
            for j in range(R // L):
                ia = raw_v[pl.ds((2 * p) * R + j * L, L)]
                ib = raw_v[pl.ds((2 * p + 1) * R + j * L, L)]
                cidx_v[pl.ds(p * R + j * L, L)] = ia * vb + ib

        def fire(c, b):
            return [pltpu.async_copy(
                tables[p].at[cidx_v.at[pl.ds(p * R + c * C, C)]],
                asm_v.at[b, :, pl.ds(p * PW, PW)], gsems[b])
                for p in range(NP)]

        gath = fire(0, 0)
        writes = [None] * NCH
        for c in range(NCH):
            b = c % 2
            for cp in gath:
                cp.wait()
            if c + 1 < NCH:
                if c - 1 >= 0:
                    writes[c - 1].wait()
                gath = fire(c + 1, 1 - b)
            writes[c] = pltpu.async_copy(
                asm_v.at[b], out.at[pl.ds(base + c * C, C), :], wsems[b])
        writes[NCH - 1].wait()
        writes[NCH - 2].wait()

    return sck(dayofweek.astype(jnp.int32), time.astype(jnp.int32),
               sex.astype(jnp.int32), age.astype(jnp.int32),
               month.astype(jnp.int32), day.astype(jnp.int32),
               T0, T1, T2)
